# Initial kernel scaffold; baseline (speedup 1.0000x reference)
import jax
import jax.numpy as jnp
from jax.experimental import pallas as pl


def kernel(x, edge_index, W, b):
    raise NotImplementedError("write your pallas kernel here")


# speedup vs baseline: 11.1450x; 11.1450x over previous
import functools

import jax
import jax.numpy as jnp
from jax import lax
from jax.experimental import pallas as pl
from jax.experimental.pallas import tpu as pltpu
from jax.experimental.pallas import tpu_sc as plsc

N = 10000
NPAD = 10240
D = 128
NE = 320000
NC, NS = 2, 16
NW = NC * NS
CH = 128
NCH = 80
EPW = CH * NCH
NEP = NW * EPW
RPT = NPAD // NS

def _vec16(val):
    return (lax.iota(jnp.int32, 16) * 0).astype(jnp.float32) + val


def _mesh():
    return plsc.VectorSubcoreMesh(
        core_axis_name="c", subcore_axis_name="s", num_cores=NC, num_subcores=NS
    )


def _deg_body(dst_hbm, out_hbm, idx_v, ones_v, zb_v, acc_sh):
    c = lax.axis_index("c")
    s = lax.axis_index("s")
    w = c * NS + s

    zero16 = _vec16(0.0)
    one16 = _vec16(1.0)

    def _fill(i, _):
        zb_v[pl.ds(i * 16, 16)] = zero16
        return 0

    lax.fori_loop(0, RPT // 16, _fill, 0)

    def _fill1(i, _):
        ones_v[pl.ds(i * 16, 16)] = one16
        return 0

    lax.fori_loop(0, CH // 16, _fill1, 0)

    pltpu.sync_copy(zb_v, acc_sh.at[pl.ds(s * RPT, RPT)])
    pltpu.sync_copy(dst_hbm.at[w], idx_v)
    plsc.subcore_barrier()

    def _step(j, _):
        pltpu.sync_copy(ones_v, acc_sh.at[idx_v.at[j]], add=True)
        return 0

    lax.fori_loop(0, NCH, _step, 0)
    plsc.subcore_barrier()
    pltpu.sync_copy(acc_sh.at[pl.ds(s * RPT, RPT)], out_hbm.at[c, pl.ds(s * RPT, RPT)])


_deg_fn = pl.kernel(
    _deg_body,
    out_type=jax.ShapeDtypeStruct((NC, NPAD), jnp.float32),
    mesh=_mesh(),
    scratch_types=[
        pltpu.VMEM((NCH, CH), jnp.int32),
        pltpu.VMEM((CH,), jnp.float32),
        pltpu.VMEM((RPT,), jnp.float32),
        pltpu.VMEM_SHARED((NPAD,), jnp.float32),
    ],
)


def _agg_body(g_hbm, src_hbm, dst_hbm, out_hbm, sidx, didx, rows, acc_sh):
    c = lax.axis_index("c")
    s = lax.axis_index("s")
    w = c * NS + s

    zero16 = _vec16(0.0)

    def _fill(i, _):
        rows[i // 8, pl.ds((i % 8) * 16, 16)] = zero16
        return 0

    lax.fori_loop(0, CH * (D // 16), _fill, 0)

    def _z(i, _):
        pltpu.sync_copy(rows, acc_sh.at[pl.ds(s * RPT + i * CH, CH)])
        return 0

    lax.fori_loop(0, RPT // CH, _z, 0)
    pltpu.sync_copy(src_hbm.at[w], sidx)
    pltpu.sync_copy(dst_hbm.at[w], didx)
    plsc.subcore_barrier()

    def _step(j, _):
        pltpu.sync_copy(g_hbm.at[sidx.at[j]], rows)
        pltpu.sync_copy(rows, acc_sh.at[didx.at[j]], add=True)
        return 0

    lax.fori_loop(0, NCH, _step, 0)
    plsc.subcore_barrier()
    pltpu.sync_copy(
        acc_sh.at[pl.ds(s * RPT, RPT)], out_hbm.at[c, pl.ds(s * RPT, RPT)]
    )


_agg_fn = pl.kernel(
    _agg_body,
    out_type=jax.ShapeDtypeStruct((NC, NPAD, D), jnp.float32),
    mesh=_mesh(),
    scratch_types=[
        pltpu.VMEM((NCH, CH), jnp.int32),
        pltpu.VMEM((NCH, CH), jnp.int32),
        pltpu.VMEM((CH, D), jnp.float32),
        pltpu.VMEM_SHARED((NPAD, D), jnp.float32),
    ],
)


RB = 1024


def _mm_body(x_ref, w_ref, d0_ref, d1_ref, g_ref, dis_ref):
    deg = d0_ref[...] + d1_ref[...] + 1.0
    dis = lax.rsqrt(deg)
    h = jnp.dot(x_ref[...], w_ref[...], preferred_element_type=jnp.float32)
    g_ref[...] = h * dis[:, None]
    dis_ref[...] = dis


_mm_fn = pl.pallas_call(
    _mm_body,
    grid=(NPAD // RB,),
    in_specs=[
        pl.BlockSpec((RB, D), lambda i: (i, 0)),
        pl.BlockSpec((D, D), lambda i: (0, 0)),
        pl.BlockSpec((RB,), lambda i: (i,)),
        pl.BlockSpec((RB,), lambda i: (i,)),
    ],
    out_specs=[
        pl.BlockSpec((RB, D), lambda i: (i, 0)),
        pl.BlockSpec((RB,), lambda i: (i,)),
    ],
    out_shape=[
        jax.ShapeDtypeStruct((NPAD, D), jnp.float32),
        jax.ShapeDtypeStruct((NPAD,), jnp.float32),
    ],
)


def _epi_body(a0_ref, a1_ref, g_ref, dis_ref, b_ref, out_ref):
    v = (a0_ref[...] + a1_ref[...] + g_ref[...]) * dis_ref[...][:, None]
    v = v + b_ref[...][None, :]
    r = jnp.maximum(v, 0.0)
    m = jnp.max(r, axis=1, keepdims=True)
    ids = lax.broadcasted_iota(jnp.int32, r.shape, 1)
    out_ref[...] = jnp.min(jnp.where(r == m, ids, D), axis=1).astype(jnp.int32)


_epi_fn = pl.pallas_call(
    _epi_body,
    grid=(NPAD // RB,),
    in_specs=[
        pl.BlockSpec((RB, D), lambda i: (i, 0)),
        pl.BlockSpec((RB, D), lambda i: (i, 0)),
        pl.BlockSpec((RB, D), lambda i: (i, 0)),
        pl.BlockSpec((RB,), lambda i: (i,)),
        pl.BlockSpec((D,), lambda i: (0,)),
    ],
    out_specs=pl.BlockSpec((RB,), lambda i: (i,)),
    out_shape=jax.ShapeDtypeStruct((NPAD,), jnp.int32),
)


def kernel(x, edge_index, W, b):
    ei = edge_index.astype(jnp.int32)
    pad = jnp.full((NEP - NE,), N, jnp.int32)
    src = jnp.concatenate([ei[0], pad]).reshape(NW, NCH, CH)
    dst = jnp.concatenate([ei[1], pad]).reshape(NW, NCH, CH)
    x_p = jnp.pad(x, ((0, NPAD - N), (0, 0)))
    deg2 = _deg_fn(dst)
    g, dis = _mm_fn(x_p, W, deg2[0], deg2[1])
    acc2 = _agg_fn(g, src, dst)
    out = _epi_fn(acc2[0], acc2[1], g, dis, b)
    return out[:N]
